# Initial kernel scaffold; baseline (speedup 1.0000x reference)
import jax
import jax.numpy as jnp
from jax.experimental import pallas as pl


def kernel(x, w1, b1, w2, b2, w3, b3, w4, b4, E, dw1, db1, dw2, db2, dw3, db3, dw4, db4):
    raise NotImplementedError("write your pallas kernel here")


# speedup vs baseline: 1.3557x; 1.3557x over previous
import jax
import jax.numpy as jnp
from jax import lax
from jax.experimental import pallas as pl
from jax.experimental.pallas import tpu as pltpu

_B = 16
_L = 2048
_D = 64
_K = 1024
_CHUNK = 512
_N_TOK = _B * _L * _D
_PREC = lax.Precision.DEFAULT


def _relu(a):
    return jnp.maximum(a, 0.0)


def _sd(a):
    return jnp.concatenate([jnp.zeros((1, a.shape[1]), jnp.float32), a[:-1]], axis=0)


def _su(a):
    return jnp.concatenate([a[1:], jnp.zeros((1, a.shape[1]), jnp.float32)], axis=0)


def _dot(a, b):
    return jnp.dot(a, b, preferred_element_type=jnp.float32)


def _vqdec(z_ref, A_ref, E_ref, ET_ref, DW1_ref, db1_ref, V_ref, db2_ref,
           G_ref, db3_ref, M4_ref, c0_ref, db4_ref,
           ph_ref, last_ref, idx_ref, loss_ref, perp_ref,
           q_s, hist_s, sse_s):
    b = pl.program_id(0)

    @pl.when(b == 0)
    def _init():
        hist_s[...] = jnp.zeros_like(hist_s)
        sse_s[...] = jnp.zeros_like(sse_s)

    z = z_ref[...]
    A = A_ref[...]
    ET = ET_ref[...]
    Emat = E_ref[...]
    Bvec = jnp.sum(Emat * Emat, axis=1)

    hist_acc = jnp.zeros((_K,), jnp.float32)
    sse_acc = jnp.zeros((1, 1), jnp.float32)
    for c in range(_L // _CHUNK):
        sl = slice(c * _CHUNK, (c + 1) * _CHUNK)
        zc = z[sl]
        Ac = A[sl]
        M = lax.dot_general(zc, ET, (((1,), (0,)), ((), ())),
                            precision=_PREC, preferred_element_type=jnp.float32)
        d = (Ac + Bvec[None, :]) - 2.0 * M
        dmin = jnp.min(d, axis=1, keepdims=True)
        iot = jax.lax.broadcasted_iota(jnp.int32, (_CHUNK, _K), 1)
        idxc = jnp.min(jnp.where(d == dmin, iot, _K), axis=1).astype(jnp.int32)
        oh = (idxc[:, None] == iot).astype(jnp.float32)
        qc = _dot(oh, Emat)
        q_s[sl, :] = zc + (qc - zc)
        idx_ref[0, sl, :] = idxc[:, None]
        hist_acc = hist_acc + jnp.sum(oh, axis=0)
        err = qc - zc
        sse_acc = sse_acc + jnp.sum(jnp.sum(err * err, axis=1, keepdims=True),
                                    axis=0, keepdims=True)
    hist_s[...] = hist_s[...] + hist_acc[None, :]
    sse_s[...] = sse_s[...] + sse_acc

    q = q_s[...]
    D1 = _relu(_dot(q, DW1_ref[...]) + db1_ref[...])
    D2 = _relu(_dot(_sd(D1), V_ref[0]) + _dot(D1, V_ref[1]) +
               _dot(_su(D1), V_ref[2]) + db2_ref[...])
    Ev = _relu(_dot(_sd(D2), G_ref[0]) + _dot(D2, G_ref[2]) + db3_ref[...])
    Od = _relu(_dot(D2, G_ref[1]) + _dot(_su(D2), G_ref[3]) + db3_ref[...])
    PH = (_dot(_sd(Od), M4_ref[0]) + _dot(Ev, M4_ref[1]) +
          _dot(Od, M4_ref[2]) + _dot(_su(Ev), M4_ref[3]) + db4_ref[...])
    ph_ref[0] = PH
    lastv = jnp.sum(Od[_L - 1:_L, :] * c0_ref[...], axis=1, keepdims=True)
    last_ref[0] = lastv + db4_ref[...]

    @pl.when(b == _B - 1)
    def _fin():
        loss_ref[...] = 1.25 * sse_s[...] / float(_N_TOK)
        p = hist_s[...] / float(_B * _L)
        ent = jnp.sum(jnp.sum(p * jnp.log(p + 1e-10), axis=1, keepdims=True),
                      axis=0, keepdims=True)
        perp_ref[...] = jnp.exp(-ent)


def _conv1d(x, w, b, stride, pad):
    y = lax.conv_general_dilated(x, w, (stride,), [(pad, pad)],
                                 dimension_numbers=('NCH', 'OIH', 'NCH'))
    return y + b[None, :, None]


def kernel(x, w1, b1, w2, b2, w3, b3, w4, b4, E,
           dw1, db1, dw2, db2, dw3, db3, dw4, db4):
    f32 = jnp.float32

    z = jax.nn.relu(_conv1d(x, w1, b1, 2, 1))
    z = jax.nn.relu(_conv1d(z, w2, b2, 2, 1))
    z = jax.nn.relu(_conv1d(z, w3, b3, 1, 1))
    z = _conv1d(z, w4, b4, 1, 0)
    flat = jnp.transpose(z, (0, 2, 1)).reshape(_B * _L, _D)
    A = jnp.sum(flat ** 2, axis=1, keepdims=True)
    ET = E.T

    DW1 = dw1[:, :, 0]
    V = jnp.stack([dw2[:, :, 2], dw2[:, :, 1], dw2[:, :, 0]], 0)
    G = jnp.stack([dw3[:, :, 3], dw3[:, :, 2], dw3[:, :, 1], dw3[:, :, 0]], 0)
    c0 = dw4[:, 0, 3]
    c1 = dw4[:, 0, 2]
    c2 = dw4[:, 0, 1]
    c3 = dw4[:, 0, 0]
    zz = jnp.zeros((32,), f32)
    M4 = jnp.stack([jnp.stack([c0, zz, zz, zz], 1),
                    jnp.stack([c2, c1, c0, zz], 1),
                    jnp.stack([zz, c3, c2, c1], 1),
                    jnp.stack([zz, zz, zz, c3], 1)], 0)

    db1r = db1[None, :]
    db2r = db2[None, :]
    db3r = db3[None, :]
    db4r = db4[None, :]
    c0r = c0[None, :]

    def fullspec(shape):
        nd = len(shape)
        return pl.BlockSpec(shape, {2: lambda b: (0, 0),
                                    3: lambda b: (0, 0, 0)}[nd])

    in_specs = [
        pl.BlockSpec((_L, _D), lambda b: (b, 0)),
        pl.BlockSpec((_L, 1), lambda b: (b, 0)),
        fullspec((_K, _D)), fullspec((_D, _K)),
        fullspec((64, 128)), fullspec((1, 128)),
        fullspec((3, 128, 64)), fullspec((1, 64)),
        fullspec((4, 64, 32)), fullspec((1, 32)),
        fullspec((4, 32, 4)), fullspec((1, 32)), fullspec((1, 1)),
    ]
    out_specs = [
        pl.BlockSpec((1, _L, 4), lambda b: (b, 0, 0)),
        pl.BlockSpec((1, 1, 1), lambda b: (b, 0, 0)),
        pl.BlockSpec((1, _L, 1), lambda b: (b, 0, 0)),
        pl.BlockSpec((1, 1), lambda b: (0, 0)),
        pl.BlockSpec((1, 1), lambda b: (0, 0)),
    ]
    out_shapes = [
        jax.ShapeDtypeStruct((_B, _L, 4), f32),
        jax.ShapeDtypeStruct((_B, 1, 1), f32),
        jax.ShapeDtypeStruct((_B, _L, 1), jnp.int32),
        jax.ShapeDtypeStruct((1, 1), f32),
        jax.ShapeDtypeStruct((1, 1), f32),
    ]
    scratch_shapes = [
        pltpu.VMEM((_L, _D), f32),
        pltpu.VMEM((1, _K), f32),
        pltpu.VMEM((1, 1), f32),
    ]

    ph, last, idxo, loss, perp = pl.pallas_call(
        _vqdec,
        grid=(_B,),
        in_specs=in_specs,
        out_specs=out_specs,
        out_shape=out_shapes,
        scratch_shapes=scratch_shapes,
        compiler_params=pltpu.CompilerParams(
            dimension_semantics=("arbitrary",)),
    )(flat, A, E, ET, DW1, db1r, V, db2r, G, db3r, M4, c0r, db4r)

    x_recon = jnp.concatenate([ph.reshape(_B, 1, _L * 4),
                               last.reshape(_B, 1, 1)], axis=2)
    indices = idxo.reshape(_B, _L)
    return (loss.reshape(()), x_recon, perp.reshape(()), indices)


# speedup vs baseline: 1.3791x; 1.0173x over previous
import jax
import jax.numpy as jnp
from jax import lax
from jax.experimental import pallas as pl
from jax.experimental.pallas import tpu as pltpu

_B = 16
_L = 2048
_D = 64
_K = 1024
_CHUNK = 512
_N_TOK = _B * _L * _D
_PREC = lax.Precision.DEFAULT


def _relu(a):
    return jnp.maximum(a, 0.0)


def _sd(a):
    return jnp.concatenate([jnp.zeros((1, a.shape[1]), jnp.float32), a[:-1]], axis=0)


def _su(a):
    return jnp.concatenate([a[1:], jnp.zeros((1, a.shape[1]), jnp.float32)], axis=0)


def _dot(a, b):
    return jnp.dot(a, b, preferred_element_type=jnp.float32)


def _vqdec(z_ref, E_ref, ET_ref, DW1_ref, db1_ref, V_ref, db2_ref,
           G_ref, db3_ref, M4_ref, c0_ref, db4_ref,
           ph_ref, last_ref, idx_ref, loss_ref, perp_ref,
           q_s, hist_s, sse_s):
    b = pl.program_id(0)

    @pl.when(b == 0)
    def _init():
        hist_s[...] = jnp.zeros_like(hist_s)
        sse_s[...] = jnp.zeros_like(sse_s)

    z = jnp.transpose(z_ref[0], (1, 0))
    ET = ET_ref[...]
    Emat = E_ref[...]
    Bvec = jnp.sum(Emat * Emat, axis=1)

    hist_acc = jnp.zeros((1, _K), jnp.float32)
    sse_acc = jnp.zeros((1, 1), jnp.float32)
    for c in range(_L // _CHUNK):
        sl = slice(c * _CHUNK, (c + 1) * _CHUNK)
        zc = z[sl]
        Ac = jnp.sum(zc * zc, axis=1, keepdims=True)
        M = lax.dot_general(zc, ET, (((1,), (0,)), ((), ())),
                            precision=_PREC, preferred_element_type=jnp.float32)
        d = (Ac + Bvec[None, :]) + M
        dmin = jnp.min(d, axis=1, keepdims=True)
        iot = jax.lax.broadcasted_iota(jnp.int32, (_CHUNK, _K), 1)
        idxc = jnp.min(jnp.where(d == dmin, iot, _K), axis=1).astype(jnp.int32)
        oh = (idxc[:, None] == iot).astype(jnp.float32)
        qc = _dot(oh, Emat)
        q_s[sl, :] = zc + (qc - zc)
        idx_ref[0, sl, :] = idxc[:, None]
        hist_acc = hist_acc + _dot(jnp.ones((1, _CHUNK), jnp.float32), oh)
        err = qc - zc
        sse_acc = sse_acc + jnp.sum(jnp.sum(err * err, axis=1, keepdims=True),
                                    axis=0, keepdims=True)
    hist_s[...] = hist_s[...] + hist_acc
    sse_s[...] = sse_s[...] + sse_acc

    q = q_s[...]
    D1 = _relu(_dot(q, DW1_ref[...]) + db1_ref[...])
    D2 = _relu(_sd(_dot(D1, V_ref[0])) + _dot(D1, V_ref[1]) +
               _su(_dot(D1, V_ref[2])) + db2_ref[...])
    Ev = _relu(_sd(_dot(D2, G_ref[0])) + _dot(D2, G_ref[2]) + db3_ref[...])
    Od = _relu(_dot(D2, G_ref[1]) + _su(_dot(D2, G_ref[3])) + db3_ref[...])
    PH = (_sd(_dot(Od, M4_ref[0])) + _dot(Ev, M4_ref[1]) +
          _dot(Od, M4_ref[2]) + _su(_dot(Ev, M4_ref[3])) + db4_ref[...])
    ph_ref[0] = PH
    lastv = jnp.sum(Od[_L - 1:_L, :] * c0_ref[...], axis=1, keepdims=True)
    last_ref[0] = lastv + db4_ref[...]

    @pl.when(b == _B - 1)
    def _fin():
        loss_ref[...] = 1.25 * sse_s[...] / float(_N_TOK)
        p = hist_s[...] / float(_B * _L)
        ent = jnp.sum(jnp.sum(p * jnp.log(p + 1e-10), axis=1, keepdims=True),
                      axis=0, keepdims=True)
        perp_ref[...] = jnp.exp(-ent)


def _conv1d(x, w, b, stride, pad):
    y = lax.conv_general_dilated(x, w, (stride,), [(pad, pad)],
                                 dimension_numbers=('NCH', 'OIH', 'NCH'))
    return y + b[None, :, None]


def kernel(x, w1, b1, w2, b2, w3, b3, w4, b4, E,
           dw1, db1, dw2, db2, dw3, db3, dw4, db4):
    f32 = jnp.float32

    z = jax.nn.relu(_conv1d(x, w1, b1, 2, 1))
    z = jax.nn.relu(_conv1d(z, w2, b2, 2, 1))
    z = jax.nn.relu(_conv1d(z, w3, b3, 1, 1))
    z = _conv1d(z, w4, b4, 1, 0)
    ET = -2.0 * E.T

    DW1 = dw1[:, :, 0]
    V = jnp.stack([dw2[:, :, 2], dw2[:, :, 1], dw2[:, :, 0]], 0)
    G = jnp.stack([dw3[:, :, 3], dw3[:, :, 2], dw3[:, :, 1], dw3[:, :, 0]], 0)
    c0 = dw4[:, 0, 3]
    c1 = dw4[:, 0, 2]
    c2 = dw4[:, 0, 1]
    c3 = dw4[:, 0, 0]
    zz = jnp.zeros((32,), f32)
    M4 = jnp.stack([jnp.stack([c0, zz, zz, zz], 1),
                    jnp.stack([c2, c1, c0, zz], 1),
                    jnp.stack([zz, c3, c2, c1], 1),
                    jnp.stack([zz, zz, zz, c3], 1)], 0)

    db1r = db1[None, :]
    db2r = db2[None, :]
    db3r = db3[None, :]
    db4r = db4[None, :]
    c0r = c0[None, :]

    def fullspec(shape):
        nd = len(shape)
        return pl.BlockSpec(shape, {2: lambda b: (0, 0),
                                    3: lambda b: (0, 0, 0)}[nd])

    in_specs = [
        pl.BlockSpec((1, _D, _L), lambda b: (b, 0, 0)),
        fullspec((_K, _D)), fullspec((_D, _K)),
        fullspec((64, 128)), fullspec((1, 128)),
        fullspec((3, 128, 64)), fullspec((1, 64)),
        fullspec((4, 64, 32)), fullspec((1, 32)),
        fullspec((4, 32, 4)), fullspec((1, 32)), fullspec((1, 1)),
    ]
    out_specs = [
        pl.BlockSpec((1, _L, 4), lambda b: (b, 0, 0)),
        pl.BlockSpec((1, 1, 1), lambda b: (b, 0, 0)),
        pl.BlockSpec((1, _L, 1), lambda b: (b, 0, 0)),
        pl.BlockSpec((1, 1), lambda b: (0, 0)),
        pl.BlockSpec((1, 1), lambda b: (0, 0)),
    ]
    out_shapes = [
        jax.ShapeDtypeStruct((_B, _L, 4), f32),
        jax.ShapeDtypeStruct((_B, 1, 1), f32),
        jax.ShapeDtypeStruct((_B, _L, 1), jnp.int32),
        jax.ShapeDtypeStruct((1, 1), f32),
        jax.ShapeDtypeStruct((1, 1), f32),
    ]
    scratch_shapes = [
        pltpu.VMEM((_L, _D), f32),
        pltpu.VMEM((1, _K), f32),
        pltpu.VMEM((1, 1), f32),
    ]

    ph, last, idxo, loss, perp = pl.pallas_call(
        _vqdec,
        grid=(_B,),
        in_specs=in_specs,
        out_specs=out_specs,
        out_shape=out_shapes,
        scratch_shapes=scratch_shapes,
        compiler_params=pltpu.CompilerParams(
            dimension_semantics=("arbitrary",)),
    )(z, E, ET, DW1, db1r, V, db2r, G, db3r, M4, c0r, db4r)

    x_recon = jnp.concatenate([ph.reshape(_B, 1, _L * 4),
                               last.reshape(_B, 1, 1)], axis=2)
    indices = idxo.reshape(_B, _L)
    return (loss.reshape(()), x_recon, perp.reshape(()), indices)
